# baseline (speedup 1.0000x reference)
import functools
import math

import jax
import jax.numpy as jnp
from jax import lax
from jax.experimental import pallas as pl
from jax.experimental.pallas import tpu as pltpu
from jax.experimental.pallas import tpu_sc as plsc

D_MODEL = 64
LANES = 16
NUM_WORKERS = 32
IDX_MINOR = 128
GATHERS_PER_CHUNK = 4
CHUNK = IDX_MINOR * GATHERS_PER_CHUNK
SCALE = math.sqrt(D_MODEL)


def _make_sc_lookup(batch, d_model):
    assert d_model == D_MODEL
    assert batch % (NUM_WORKERS * CHUNK) == 0
    rows_per_w = batch // NUM_WORKERS
    idx_rows_per_w = rows_per_w // IDX_MINOR
    n_chunks = rows_per_w // CHUNK

    mesh = plsc.VectorSubcoreMesh(core_axis_name="c", subcore_axis_name="s")

    @functools.partial(
        pl.kernel,
        mesh=mesh,
        out_type=jax.ShapeDtypeStruct((batch, d_model), jnp.float32),
        compiler_params=pltpu.CompilerParams(use_tc_tiling_on_sc=False),
        scratch_types=[
            pltpu.VMEM((idx_rows_per_w, IDX_MINOR), jnp.int32),
            pltpu.VMEM((CHUNK, D_MODEL), jnp.float32),
            pltpu.VMEM((CHUNK, D_MODEL), jnp.float32),
            pltpu.SemaphoreType.DMA,
            pltpu.SemaphoreType.DMA,
        ],
    )
    def sc_lookup(idx_hbm, table_hbm, out_hbm, idx_v, rows0, rows1, sem0, sem1):
        wid = lax.axis_index("s") * 2 + lax.axis_index("c")
        idx_row_base = wid * idx_rows_per_w
        out_base = wid * rows_per_w

        rows = (rows0, rows1)
        sems = (sem0, sem1)

        pltpu.sync_copy(idx_hbm.at[pl.ds(idx_row_base, idx_rows_per_w)], idx_v)

        def fire(chunk, buf):
            for g in range(GATHERS_PER_CHUNK):
                pltpu.async_copy(
                    table_hbm.at[idx_v.at[chunk * GATHERS_PER_CHUNK + g]],
                    rows[buf].at[pl.ds(g * IDX_MINOR, IDX_MINOR)],
                    sems[buf],
                )

        def drain(chunk, buf):
            for g in range(GATHERS_PER_CHUNK):
                pltpu.make_async_copy(
                    table_hbm.at[idx_v.at[chunk * GATHERS_PER_CHUNK + g]],
                    rows[buf].at[pl.ds(g * IDX_MINOR, IDX_MINOR)],
                    sems[buf],
                ).wait()

        fire(0, 0)
        fire(1, 1)

        def chunk_body(i, carry):
            for buf in range(2):
                c = 2 * i + buf
                drain(c, buf)

                def scale_body(r, acc):
                    for rr in range(4):
                        for s in range(D_MODEL // LANES):
                            sl = (4 * r + rr, pl.ds(s * LANES, LANES))
                            rows[buf][sl] = rows[buf][sl] * SCALE
                    return acc

                lax.fori_loop(0, CHUNK // 4, scale_body, 0)

                pltpu.sync_copy(
                    rows[buf],
                    out_hbm.at[pl.ds(out_base + c * CHUNK, CHUNK)],
                )

                @pl.when(c + 2 < n_chunks)
                def _():
                    fire(c + 2, buf)
            return carry

        lax.fori_loop(0, n_chunks // 2, chunk_body, 0)

    return sc_lookup


def kernel(x, table):
    batch = x.shape[0] * x.shape[1]
    xi = x.reshape(batch).astype(jnp.int32).reshape(batch // IDX_MINOR, IDX_MINOR)
    out = _make_sc_lookup(batch, table.shape[1])(xi, table)
    return out.reshape(x.shape[0], x.shape[1], D_MODEL)
